# speedup vs baseline: 3.1828x; 3.1828x over previous
import functools

import jax
import jax.numpy as jnp
from jax.experimental import pallas as pl
from jax.experimental.pallas import tpu as pltpu
from jax.experimental.pallas import tpu_sc as plsc

_WINDOW = 128


def _build_matrix(r, E, l, h):
    V, W = E.shape

    def body(r_ref, e_ref, l_ref, h_ref, o_ref):
        rr = r_ref[...]
        o_ref[...] = e_ref[...] + rr * l_ref[...] + (1.0 - rr) * h_ref[...]

    return pl.pallas_call(
        body,
        out_shape=jax.ShapeDtypeStruct((V, W), jnp.float32),
    )(r, E, l.reshape(1, W), h.reshape(1, W))


def kernel(idx, r, E, l, h):
    V, W = E.shape
    B, F = idx.shape
    BF = B * F
    assert BF % _WINDOW == 0

    matrix = _build_matrix(r, E, l, h)
    idx_flat = idx.reshape(1, BF).astype(jnp.int32)

    mesh = plsc.VectorSubcoreMesh(
        core_axis_name="core", subcore_axis_name="subcore"
    )

    @functools.partial(
        pl.kernel,
        out_type=jax.ShapeDtypeStruct((BF, W), jnp.float32),
        mesh=mesh,
    )
    def gather_k(x_hbm, i_hbm, o_hbm):
        def body(i_vmem, o_vmem):
            pltpu.sync_copy(x_hbm.at[i_vmem.at[0]], o_vmem)

        pltpu.emit_pipeline(
            body,
            grid=(BF // _WINDOW,),
            in_specs=[pl.BlockSpec((1, _WINDOW), index_map=lambda i: (0, i))],
            out_specs=[
                pl.BlockSpec((_WINDOW, W), index_map=lambda i: (i, 0))
            ],
            core_axis_name=("core", "subcore"),
            dimension_semantics=(pltpu.PARALLEL,),
        )(i_hbm, o_hbm)

    out = gather_k(matrix, idx_flat)
    return out.reshape(B, F, W)


# speedup vs baseline: 5.2891x; 1.6618x over previous
import functools

import jax
import jax.numpy as jnp
from jax.experimental import pallas as pl
from jax.experimental.pallas import tpu as pltpu
from jax.experimental.pallas import tpu_sc as plsc

_WINDOW = 128


def _build_matrix(r, E, l, h):
    V, W = E.shape

    def body(r_ref, e_ref, l_ref, h_ref, o_ref):
        rr = r_ref[...]
        o_ref[...] = e_ref[...] + rr * l_ref[...] + (1.0 - rr) * h_ref[...]

    return pl.pallas_call(
        body,
        out_shape=jax.ShapeDtypeStruct((V, W), jnp.float32),
    )(r, E, l.reshape(1, W), h.reshape(1, W))


def kernel(idx, r, E, l, h):
    V, W = E.shape
    B, F = idx.shape
    BW = 4
    assert B % BW == 0

    matrix = _build_matrix(r, E, l, h)
    idx32 = idx.astype(jnp.int32)

    mesh = plsc.VectorSubcoreMesh(
        core_axis_name="core", subcore_axis_name="subcore"
    )

    @functools.partial(
        pl.kernel,
        out_type=jax.ShapeDtypeStruct((B, F, W), jnp.float32),
        mesh=mesh,
    )
    def gather_k(x_hbm, i_hbm, o_hbm):
        def body(i_vmem, o_vmem):
            for b in range(BW):
                pltpu.sync_copy(x_hbm.at[i_vmem.at[b]], o_vmem.at[b])

        pltpu.emit_pipeline(
            body,
            grid=(B // BW,),
            in_specs=[pl.BlockSpec((BW, F), index_map=lambda i: (i, 0))],
            out_specs=[
                pl.BlockSpec((BW, F, W), index_map=lambda i: (i, 0, 0))
            ],
            core_axis_name=("core", "subcore"),
            dimension_semantics=(pltpu.PARALLEL,),
        )(i_hbm, o_hbm)

    return gather_k(matrix, idx32)


# speedup vs baseline: 8.9273x; 1.6879x over previous
import functools

import jax
import jax.numpy as jnp
from jax.experimental import pallas as pl
from jax.experimental.pallas import tpu as pltpu
from jax.experimental.pallas import tpu_sc as plsc

_WINDOW = 128


def _build_matrix(r, E, l, h):
    V, W = E.shape

    def body(r_ref, e_ref, l_ref, h_ref, o_ref):
        rr = r_ref[...]
        o_ref[...] = e_ref[...] + rr * l_ref[...] + (1.0 - rr) * h_ref[...]

    return pl.pallas_call(
        body,
        out_shape=jax.ShapeDtypeStruct((V, W), jnp.float32),
    )(r, E, l.reshape(1, W), h.reshape(1, W))


def kernel(idx, r, E, l, h):
    V, W = E.shape
    B, F = idx.shape
    BW = 4
    assert B % BW == 0

    matrix = _build_matrix(r, E, l, h)
    idx32 = idx.astype(jnp.int32)

    mesh = plsc.VectorSubcoreMesh(
        core_axis_name="core", subcore_axis_name="subcore"
    )

    @functools.partial(
        pl.kernel,
        out_type=jax.ShapeDtypeStruct((B, F, W), jnp.float32),
        mesh=mesh,
        scratch_types=[pltpu.VMEM_SHARED((V, W), jnp.float32)],
    )
    def gather_k(x_hbm, i_hbm, o_hbm, tbl_sh):
        @pl.when(jax.lax.axis_index("subcore") == 0)
        def _():
            pltpu.sync_copy(x_hbm, tbl_sh)

        plsc.subcore_barrier()

        def body(i_vmem, o_vmem):
            for b in range(BW):
                pltpu.sync_copy(tbl_sh.at[i_vmem.at[b]], o_vmem.at[b])

        pltpu.emit_pipeline(
            body,
            grid=(B // BW,),
            in_specs=[pl.BlockSpec((BW, F), index_map=lambda i: (i, 0))],
            out_specs=[
                pl.BlockSpec((BW, F, W), index_map=lambda i: (i, 0, 0))
            ],
            core_axis_name=("core", "subcore"),
            dimension_semantics=(pltpu.PARALLEL,),
        )(i_hbm, o_hbm)

    return gather_k(matrix, idx32)
